# baseline (speedup 1.0000x reference)
import functools

import jax
import jax.numpy as jnp
from jax.experimental import pallas as pl
from jax.experimental.pallas import tpu as pltpu

B, C, H, W = 2, 96, 384, 384
QK_DIM = 96
NUM_HEADS = 3
K_SP = 576
TOPK = 64
HEAD_DIM = QK_DIM // NUM_HEADS
SCALE = HEAD_DIM ** (-0.5)
HW = H * W
NSP = B * K_SP


def _attn_body(xg_ref, sims_ref, qw_ref, kw_ref, vw_ref, lnw_ref, lnb_ref, out_ref):
    xg = xg_ref[...]
    u = jnp.mean(xg, axis=1, keepdims=True)
    var = jnp.mean((xg - u) ** 2, axis=1, keepdims=True)
    xn = (xg - u) * jax.lax.rsqrt(var + 1e-6)
    xn = xn * lnw_ref[...] + lnb_ref[...]
    dn = (((1,), (1,)), ((), ()))
    q = jax.lax.dot_general(xn, qw_ref[...], dn, preferred_element_type=jnp.float32)
    k = jax.lax.dot_general(xn, kw_ref[...], dn, preferred_element_type=jnp.float32)
    v = jax.lax.dot_general(xn, vw_ref[...], dn, preferred_element_type=jnp.float32)
    s_col = sims_ref[...].reshape(TOPK, 1)
    ones_row = jnp.ones((1, HEAD_DIM), dtype=jnp.float32)
    for h in range(NUM_HEADS):
        qh = q[:, h * HEAD_DIM:(h + 1) * HEAD_DIM]
        kh = k[:, h * HEAD_DIM:(h + 1) * HEAD_DIM]
        vh = v[:, h * HEAD_DIM:(h + 1) * HEAD_DIM]
        qq = jnp.sum(qh * qh, axis=1, keepdims=True)
        kk = jax.lax.dot_general(ones_row, kh * kh, dn,
                                 preferred_element_type=jnp.float32)
        qk = jax.lax.dot_general(qh, kh, dn, preferred_element_type=jnp.float32)
        d2 = qq + kk - 2.0 * qk
        dist = jnp.sqrt(jnp.maximum(d2, 1e-12))
        a = -SCALE * dist
        m = jnp.max(a, axis=1, keepdims=True)
        e = jnp.exp(a - m)
        p = e / jnp.sum(e, axis=1, keepdims=True)
        vw_h = s_col * vh
        oh = jax.lax.dot_general(p, vw_h, (((1,), (0,)), ((), ())),
                                 preferred_element_type=jnp.float32)
        out_ref[:, h * HEAD_DIM:(h + 1) * HEAD_DIM] = s_col * oh


def _attention(xg2, simsT, q_w, k_w, v_w, ln_w, ln_b):
    return pl.pallas_call(
        _attn_body,
        grid=(NSP,),
        in_specs=[
            pl.BlockSpec((TOPK, C), lambda i: (i, 0)),
            pl.BlockSpec((1, TOPK, 1), lambda i: (i, 0, 0)),
            pl.BlockSpec((QK_DIM, C), lambda i: (0, 0)),
            pl.BlockSpec((QK_DIM, C), lambda i: (0, 0)),
            pl.BlockSpec((C, C), lambda i: (0, 0)),
            pl.BlockSpec((1, C), lambda i: (0, 0)),
            pl.BlockSpec((1, C), lambda i: (0, 0)),
        ],
        out_specs=pl.BlockSpec((TOPK, C), lambda i: (i, 0)),
        out_shape=jax.ShapeDtypeStruct((NSP * TOPK, C), jnp.float32),
    )(xg2, simsT, q_w, k_w, v_w, ln_w, ln_b)


def kernel(x, sims, mask, ln_w, ln_b, q_w, k_w, v_w, indices, labels, num_spixels):
    xt = x.reshape(B, C, HW).transpose(0, 2, 1)
    idx = indices.reshape(B, K_SP * TOPK)
    xg = jnp.take_along_axis(xt, idx[..., None], axis=1)

    u = jnp.mean(xt, axis=-1, keepdims=True)
    var = jnp.mean((xt - u) ** 2, axis=-1, keepdims=True)
    xn = (xt - u) / jnp.sqrt(var + 1e-6) * ln_w + ln_b
    v_full = jnp.einsum('bpc,oc->bpo', xn, v_w)

    out_tok = _attention(
        xg.reshape(B * K_SP * TOPK, C),
        sims.reshape(NSP, TOPK, 1),
        q_w, k_w, v_w, ln_w.reshape(1, C), ln_b.reshape(1, C))
    out_tok = out_tok.reshape(B, K_SP * TOPK, C)

    acc = jax.vmap(lambda s, i: jax.ops.segment_sum(s, i, num_segments=HW))(out_tok, idx)
    cnt = jax.vmap(lambda i: jax.ops.segment_sum(
        jnp.ones_like(i, jnp.float32), i, num_segments=HW))(idx)
    mean = acc / jnp.maximum(cnt[..., None], 1.0)
    merged = jnp.where(cnt[..., None] > 1e-5, mean, v_full)
    return merged.transpose(0, 2, 1).reshape(B, C, H, W)
